# baseline (speedup 1.0000x reference)
import functools

import jax
import jax.numpy as jnp
from jax import lax
from jax.experimental import pallas as pl
from jax.experimental.pallas import tpu as pltpu
from jax.experimental.pallas import tpu_sc as plsc

N_NODES = 10000
N_EDGES = 10000
NNZ = 320000
D = 128

NC = 2
NS = 16
NW = NC * NS
PER_W = NNZ // NW
LANES = 16

ROW_BLK = 1000


def _sc_gather_scatter_add(nseg: int, with_cnt: bool, k: int,
                           nbuf: int = 2):
    nchunk = PER_W // k
    h1 = (nchunk + 1) // 2
    stages = [(0, h1), (h1, nchunk - h1)]
    rpt = (nseg // NS) // 8 * 8
    tail = nseg - rpt * NS
    zc = (k // 8) * 8
    zrem = rpt % zc
    mesh = plsc.VectorSubcoreMesh(
        core_axis_name="c", subcore_axis_name="s",
        num_cores=NC, num_subcores=NS)

    out_type = [jax.ShapeDtypeStruct((NC, nseg, D), jnp.float32)]
    scratch = (
        [pltpu.VMEM((h1, k), jnp.int32),
         pltpu.VMEM((h1, k), jnp.int32)]
        + [pltpu.VMEM((k, D), jnp.float32)] * nbuf
        + [pltpu.VMEM_SHARED((nseg, D), jnp.float32)]
        + [pltpu.SemaphoreType.DMA] * nbuf
        + [pltpu.SemaphoreType.DMA] * nbuf
    )
    if with_cnt:
        out_type.append(jax.ShapeDtypeStruct((NC, nseg, LANES), jnp.float32))
        scratch.append(pltpu.VMEM((k, LANES), jnp.float32))
        scratch.append(pltpu.VMEM_SHARED((nseg, LANES), jnp.float32))
        scratch.extend([pltpu.SemaphoreType.DMA] * nbuf)

    def body(table, gidx, sidx, acc_out, *rest):
        if with_cnt:
            cnt_out = rest[0]
            rest = rest[1:]
        gidx_v, sidx_v = rest[0], rest[1]
        rows = rest[2:2 + nbuf]
        acc_sh = rest[2 + nbuf]
        sems = rest[3 + nbuf:3 + 2 * nbuf]
        semss = rest[3 + 2 * nbuf:3 + 3 * nbuf]
        if with_cnt:
            ones_v, cnt_sh = rest[3 + 3 * nbuf], rest[4 + 3 * nbuf]
            semcs = rest[5 + 3 * nbuf:5 + 4 * nbuf]
        else:
            ones_v = cnt_sh = semcs = None
        rows0 = rows[0]
        c = lax.axis_index("c")
        s = lax.axis_index("s")
        wid = s * NC + c

        zeros = jnp.zeros((LANES,), jnp.float32)

        def zfill(i, _):
            for j in range(D // LANES):
                rows0[i, pl.ds(j * LANES, LANES)] = zeros
            return _

        lax.fori_loop(0, zc, zfill, None)
        for r in range(rpt // zc):
            pltpu.sync_copy(rows0.at[pl.ds(0, zc)],
                            acc_sh.at[pl.ds(s * rpt + r * zc, zc)])
        if zrem:
            pltpu.sync_copy(rows0.at[pl.ds(0, zrem)],
                            acc_sh.at[pl.ds(s * rpt + (rpt // zc) * zc,
                                            zrem)])

        @pl.when(s == 0)
        def _zero_tail():
            pltpu.sync_copy(rows0.at[pl.ds(0, tail)],
                            acc_sh.at[pl.ds(rpt * NS, tail)])

        if with_cnt:
            def c0(i, _):
                ones_v[i, :] = zeros
                return _
            lax.fori_loop(0, k, c0, None)
            for r in range(rpt // zc):
                pltpu.sync_copy(ones_v.at[pl.ds(0, zc)],
                                cnt_sh.at[pl.ds(s * rpt + r * zc, zc)])
            if zrem:
                pltpu.sync_copy(ones_v.at[pl.ds(0, zrem)],
                                cnt_sh.at[pl.ds(s * rpt + (rpt // zc) * zc,
                                                zrem)])

            @pl.when(s == 0)
            def _czero_tail():
                pltpu.sync_copy(ones_v.at[pl.ds(0, tail)],
                                cnt_sh.at[pl.ds(rpt * NS, tail)])

            ones = jnp.ones((LANES,), jnp.float32)

            def c1(i, _):
                ones_v[i, :] = ones
                return _
            lax.fori_loop(0, k, c1, None)
        plsc.subcore_barrier()

        def do_chunk(j, n_sub):
            static = isinstance(j, int)
            b = j % nbuf if static else None
            if static:
                rb, gsem, ssem = rows[b], sems[b], semss[b]
                csem = semcs[b] if with_cnt else None
            else:
                raise AssertionError
            pltpu.make_async_copy(table.at[gidx_v.at[j]], rb, gsem).wait()
            if nbuf == 2:
                pltpu.sync_copy(rb, acc_sh.at[sidx_v.at[j]], add=True)
            else:
                pltpu.async_copy(rb, acc_sh.at[sidx_v.at[j]], ssem,
                                 add=True)
            if with_cnt:
                if j >= nbuf:
                    pltpu.make_async_copy(
                        ones_v, cnt_sh.at[sidx_v.at[j]], csem).wait()
                pltpu.async_copy(ones_v, cnt_sh.at[sidx_v.at[j]], csem,
                                 add=True)
            if j + 2 < n_sub:
                bp = (j + 2) % nbuf
                if nbuf >= 3 and j >= 1:
                    pltpu.make_async_copy(
                        rows[bp], acc_sh.at[sidx_v.at[j - 1]],
                        semss[bp]).wait()
                pltpu.async_copy(table.at[gidx_v.at[j + 2]], rows[bp],
                                 sems[bp])

        def do_chunk_traced(j, u, i, n_sub, nloop):
            rb, gsem, ssem = rows[u], sems[u], semss[u]
            pltpu.make_async_copy(table.at[gidx_v.at[j]], rb, gsem).wait()
            if nbuf == 2:
                pltpu.sync_copy(rb, acc_sh.at[sidx_v.at[j]], add=True)
            else:
                pltpu.async_copy(rb, acc_sh.at[sidx_v.at[j]], ssem,
                                 add=True)
            if with_cnt:
                csem = semcs[u]

                @pl.when(i > 0)
                def _drain_cnt():
                    pltpu.make_async_copy(
                        ones_v, cnt_sh.at[sidx_v.at[j]], csem).wait()
                pltpu.async_copy(ones_v, cnt_sh.at[sidx_v.at[j]], csem,
                                 add=True)
            bp = (u + 2) % nbuf
            guard = (j + 2 < n_sub)
            if nbuf >= 3:
                guard = guard & (j >= 1)

            @pl.when(guard)
            def _prefetch():
                if nbuf >= 3:
                    pltpu.make_async_copy(
                        rows[bp], acc_sh.at[sidx_v.at[j - 1]],
                        semss[bp]).wait()
                pltpu.async_copy(table.at[gidx_v.at[j + 2]], rows[bp],
                                 sems[bp])

        for base, n_sub in stages:
            pltpu.sync_copy(gidx.at[wid, pl.ds(base, n_sub)],
                            gidx_v.at[pl.ds(0, n_sub)])
            pltpu.sync_copy(sidx.at[wid, pl.ds(base, n_sub)],
                            sidx_v.at[pl.ds(0, n_sub)])

            for t in range(2 if nbuf == 2 else 3):
                pltpu.async_copy(table.at[gidx_v.at[t]], rows[t], sems[t])

            nloop = n_sub // nbuf

            def step(i, _):
                for u in range(nbuf):
                    do_chunk_traced(nbuf * i + u, u, i, n_sub, nloop)
                return _

            lax.fori_loop(0, nloop, step, None)
            for j in range(nloop * nbuf, n_sub):
                do_chunk(j, n_sub)

            if nbuf >= 3:
                for t in range(max(0, n_sub - 3), n_sub):
                    pltpu.make_async_copy(
                        rows[t % nbuf], acc_sh.at[sidx_v.at[t]],
                        semss[t % nbuf]).wait()
            if with_cnt:
                for t in range(max(0, n_sub - nbuf), n_sub):
                    pltpu.make_async_copy(
                        ones_v, cnt_sh.at[sidx_v.at[t]],
                        semcs[t % nbuf]).wait()
        plsc.subcore_barrier()

        pltpu.sync_copy(acc_sh.at[pl.ds(s * rpt, rpt)],
                        acc_out.at[c, pl.ds(s * rpt, rpt)])

        @pl.when(s == 0)
        def _write_tail():
            pltpu.sync_copy(acc_sh.at[pl.ds(rpt * NS, tail)],
                            acc_out.at[c, pl.ds(rpt * NS, tail)])

        if with_cnt:
            pltpu.sync_copy(cnt_sh.at[pl.ds(s * rpt, rpt)],
                            cnt_out.at[c, pl.ds(s * rpt, rpt)])

            @pl.when(s == 0)
            def _cnt_tail():
                pltpu.sync_copy(cnt_sh.at[pl.ds(rpt * NS, tail)],
                                cnt_out.at[c, pl.ds(rpt * NS, tail)])

    return pl.kernel(
        body, out_type=out_type, mesh=mesh, scratch_types=scratch,
        compiler_params=pltpu.CompilerParams(
            needs_layout_passes=False, use_tc_tiling_on_sc=False))


def _tc_matmul(X, W):
    n, d_in = X.shape
    d_out = W.shape[0]

    def body(x_ref, w_ref, o_ref):
        o_ref[...] = lax.dot_general(
            x_ref[...], w_ref[...], (((1,), (1,)), ((), ())),
            preferred_element_type=jnp.float32)

    return pl.pallas_call(
        body,
        grid=(n // ROW_BLK,),
        in_specs=[
            pl.BlockSpec((ROW_BLK, d_in), lambda i: (i, 0)),
            pl.BlockSpec((d_out, d_in), lambda i: (0, 0)),
        ],
        out_specs=pl.BlockSpec((ROW_BLK, d_out), lambda i: (i, 0)),
        out_shape=jax.ShapeDtypeStruct((n, d_out), jnp.float32),
    )(X, W)


def _tc_mean_combine(esum_part, cnt_part):
    nseg = esum_part.shape[1]

    def body(e_ref, c_ref, o_ref):
        cnt = jnp.sum(c_ref[0] + c_ref[1], axis=-1) * (1.0 / LANES)
        e = e_ref[0] + e_ref[1]
        o_ref[...] = e / jnp.maximum(cnt, 1.0)[:, None]

    return pl.pallas_call(
        body,
        grid=(nseg // ROW_BLK,),
        in_specs=[
            pl.BlockSpec((NC, ROW_BLK, D), lambda i: (0, i, 0)),
            pl.BlockSpec((NC, ROW_BLK, LANES), lambda i: (0, i, 0)),
        ],
        out_specs=pl.BlockSpec((ROW_BLK, D), lambda i: (i, 0)),
        out_shape=jax.ShapeDtypeStruct((nseg, D), jnp.float32),
    )(esum_part, cnt_part)


def _tc_final(X2, xv_part, eps):
    n = X2.shape[0]

    def body(x2_ref, v_ref, eps_ref, o_ref):
        o_ref[...] = (1.0 + eps_ref[0]) * x2_ref[...] + v_ref[0] + v_ref[1]

    return pl.pallas_call(
        body,
        grid=(n // ROW_BLK,),
        in_specs=[
            pl.BlockSpec((ROW_BLK, D), lambda i: (i, 0)),
            pl.BlockSpec((NC, ROW_BLK, D), lambda i: (0, i, 0)),
            pl.BlockSpec(memory_space=pltpu.SMEM),
        ],
        out_specs=pl.BlockSpec((ROW_BLK, D), lambda i: (i, 0)),
        out_shape=jax.ShapeDtypeStruct((n, D), jnp.float32),
    )(X2, xv_part, eps)


def kernel(X, vertex, edges, W, eps):
    X2 = _tc_matmul(X, W)

    k1, k2 = 100, 80

    esum_part, cnt_part = _sc_gather_scatter_add(N_EDGES, True, k1)(
        X2, vertex.reshape(NW, PER_W // k1, k1),
        edges.reshape(NW, PER_W // k1, k1))
    Xe = _tc_mean_combine(esum_part, cnt_part)

    (xv_part,) = _sc_gather_scatter_add(N_NODES, False, k2, nbuf=3)(
        Xe, edges.reshape(NW, PER_W // k2, k2),
        vertex.reshape(NW, PER_W // k2, k2))

    Xout = _tc_final(X2, xv_part, eps)
    return (Xout, Xe)


# speedup vs baseline: 1.0058x; 1.0058x over previous
import functools

import jax
import jax.numpy as jnp
from jax import lax
from jax.experimental import pallas as pl
from jax.experimental.pallas import tpu as pltpu
from jax.experimental.pallas import tpu_sc as plsc

N_NODES = 10000
N_EDGES = 10000
NNZ = 320000
D = 128

NC = 2
NS = 16
NW = NC * NS
PER_W = NNZ // NW
LANES = 16

ROW_BLK = 1000


def _sc_gather_scatter_add(nseg: int, with_cnt: bool, k: int,
                           nbuf: int = 2):
    nchunk = PER_W // k
    h1 = (nchunk + 1) // 2
    stages = [(0, h1), (h1, nchunk - h1)]
    rpt = (nseg // NS) // 8 * 8
    tail = nseg - rpt * NS
    zc = (k // 8) * 8
    zrem = rpt % zc
    mesh = plsc.VectorSubcoreMesh(
        core_axis_name="c", subcore_axis_name="s",
        num_cores=NC, num_subcores=NS)

    out_type = [jax.ShapeDtypeStruct((NC, nseg, D), jnp.float32)]
    scratch = (
        [pltpu.VMEM((h1, k), jnp.int32),
         pltpu.VMEM((h1, k), jnp.int32)]
        + [pltpu.VMEM((k, D), jnp.float32)] * nbuf
        + [pltpu.VMEM_SHARED((nseg, D), jnp.float32)]
        + [pltpu.SemaphoreType.DMA] * nbuf
        + [pltpu.SemaphoreType.DMA] * nbuf
    )
    if with_cnt:
        out_type.append(jax.ShapeDtypeStruct((NC, nseg, LANES), jnp.float32))
        scratch.append(pltpu.VMEM((k, LANES), jnp.float32))
        scratch.append(pltpu.VMEM_SHARED((nseg, LANES), jnp.float32))
        scratch.extend([pltpu.SemaphoreType.DMA] * nbuf)

    def body(table, gidx, sidx, acc_out, *rest):
        if with_cnt:
            cnt_out = rest[0]
            rest = rest[1:]
        gidx_v, sidx_v = rest[0], rest[1]
        rows = rest[2:2 + nbuf]
        acc_sh = rest[2 + nbuf]
        sems = rest[3 + nbuf:3 + 2 * nbuf]
        semss = rest[3 + 2 * nbuf:3 + 3 * nbuf]
        if with_cnt:
            ones_v, cnt_sh = rest[3 + 3 * nbuf], rest[4 + 3 * nbuf]
            semcs = rest[5 + 3 * nbuf:5 + 4 * nbuf]
        else:
            ones_v = cnt_sh = semcs = None
        rows0 = rows[0]
        c = lax.axis_index("c")
        s = lax.axis_index("s")
        wid = s * NC + c

        zeros = jnp.zeros((LANES,), jnp.float32)

        def zfill(i, _):
            for j in range(D // LANES):
                rows0[i, pl.ds(j * LANES, LANES)] = zeros
            return _

        lax.fori_loop(0, zc, zfill, None)
        for r in range(rpt // zc):
            pltpu.sync_copy(rows0.at[pl.ds(0, zc)],
                            acc_sh.at[pl.ds(s * rpt + r * zc, zc)])
        if zrem:
            pltpu.sync_copy(rows0.at[pl.ds(0, zrem)],
                            acc_sh.at[pl.ds(s * rpt + (rpt // zc) * zc,
                                            zrem)])

        @pl.when(s == 0)
        def _zero_tail():
            pltpu.sync_copy(rows0.at[pl.ds(0, tail)],
                            acc_sh.at[pl.ds(rpt * NS, tail)])

        if with_cnt:
            def c0(i, _):
                ones_v[i, :] = zeros
                return _
            lax.fori_loop(0, k, c0, None)
            for r in range(rpt // zc):
                pltpu.sync_copy(ones_v.at[pl.ds(0, zc)],
                                cnt_sh.at[pl.ds(s * rpt + r * zc, zc)])
            if zrem:
                pltpu.sync_copy(ones_v.at[pl.ds(0, zrem)],
                                cnt_sh.at[pl.ds(s * rpt + (rpt // zc) * zc,
                                                zrem)])

            @pl.when(s == 0)
            def _czero_tail():
                pltpu.sync_copy(ones_v.at[pl.ds(0, tail)],
                                cnt_sh.at[pl.ds(rpt * NS, tail)])

            ones = jnp.ones((LANES,), jnp.float32)

            def c1(i, _):
                ones_v[i, :] = ones
                return _
            lax.fori_loop(0, k, c1, None)
        plsc.subcore_barrier()

        def do_chunk(j, n_sub):
            static = isinstance(j, int)
            b = j % nbuf if static else None
            if static:
                rb, gsem, ssem = rows[b], sems[b], semss[b]
                csem = semcs[b] if with_cnt else None
            else:
                raise AssertionError
            pltpu.make_async_copy(table.at[gidx_v.at[j]], rb, gsem).wait()
            if nbuf == 2:
                pltpu.sync_copy(rb, acc_sh.at[sidx_v.at[j]], add=True)
            else:
                pltpu.async_copy(rb, acc_sh.at[sidx_v.at[j]], ssem,
                                 add=True)
            if with_cnt:
                if j >= nbuf:
                    pltpu.make_async_copy(
                        ones_v, cnt_sh.at[sidx_v.at[j]], csem).wait()
                pltpu.async_copy(ones_v, cnt_sh.at[sidx_v.at[j]], csem,
                                 add=True)
            if j + 2 < n_sub:
                bp = (j + 2) % nbuf
                if nbuf >= 3 and j >= 1:
                    pltpu.make_async_copy(
                        rows[bp], acc_sh.at[sidx_v.at[j - 1]],
                        semss[bp]).wait()
                pltpu.async_copy(table.at[gidx_v.at[j + 2]], rows[bp],
                                 sems[bp])

        def do_chunk_traced(j, u, i, n_sub, nloop):
            rb, gsem, ssem = rows[u], sems[u], semss[u]
            pltpu.make_async_copy(table.at[gidx_v.at[j]], rb, gsem).wait()
            if nbuf == 2:
                pltpu.sync_copy(rb, acc_sh.at[sidx_v.at[j]], add=True)
            else:
                pltpu.async_copy(rb, acc_sh.at[sidx_v.at[j]], ssem,
                                 add=True)
            if with_cnt:
                csem = semcs[u]

                @pl.when(i > 0)
                def _drain_cnt():
                    pltpu.make_async_copy(
                        ones_v, cnt_sh.at[sidx_v.at[j]], csem).wait()
                pltpu.async_copy(ones_v, cnt_sh.at[sidx_v.at[j]], csem,
                                 add=True)
            bp = (u + 2) % nbuf
            guard = (j + 2 < n_sub)
            if nbuf >= 3:
                guard = guard & (j >= 1)

            @pl.when(guard)
            def _prefetch():
                if nbuf >= 3:
                    pltpu.make_async_copy(
                        rows[bp], acc_sh.at[sidx_v.at[j - 1]],
                        semss[bp]).wait()
                pltpu.async_copy(table.at[gidx_v.at[j + 2]], rows[bp],
                                 sems[bp])

        for base, n_sub in stages:
            pltpu.sync_copy(gidx.at[wid, pl.ds(base, n_sub)],
                            gidx_v.at[pl.ds(0, n_sub)])
            pltpu.sync_copy(sidx.at[wid, pl.ds(base, n_sub)],
                            sidx_v.at[pl.ds(0, n_sub)])

            for t in range(2 if nbuf == 2 else 3):
                pltpu.async_copy(table.at[gidx_v.at[t]], rows[t], sems[t])

            nloop = n_sub // nbuf

            def step(i, _):
                for u in range(nbuf):
                    do_chunk_traced(nbuf * i + u, u, i, n_sub, nloop)
                return _

            lax.fori_loop(0, nloop, step, None)
            for j in range(nloop * nbuf, n_sub):
                do_chunk(j, n_sub)

            if nbuf >= 3:
                for t in range(max(0, n_sub - 3), n_sub):
                    pltpu.make_async_copy(
                        rows[t % nbuf], acc_sh.at[sidx_v.at[t]],
                        semss[t % nbuf]).wait()
            if with_cnt:
                for t in range(max(0, n_sub - nbuf), n_sub):
                    pltpu.make_async_copy(
                        ones_v, cnt_sh.at[sidx_v.at[t]],
                        semcs[t % nbuf]).wait()
        plsc.subcore_barrier()

        pltpu.sync_copy(acc_sh.at[pl.ds(s * rpt, rpt)],
                        acc_out.at[c, pl.ds(s * rpt, rpt)])

        @pl.when(s == 0)
        def _write_tail():
            pltpu.sync_copy(acc_sh.at[pl.ds(rpt * NS, tail)],
                            acc_out.at[c, pl.ds(rpt * NS, tail)])

        if with_cnt:
            pltpu.sync_copy(cnt_sh.at[pl.ds(s * rpt, rpt)],
                            cnt_out.at[c, pl.ds(s * rpt, rpt)])

            @pl.when(s == 0)
            def _cnt_tail():
                pltpu.sync_copy(cnt_sh.at[pl.ds(rpt * NS, tail)],
                                cnt_out.at[c, pl.ds(rpt * NS, tail)])

    return pl.kernel(
        body, out_type=out_type, mesh=mesh, scratch_types=scratch,
        compiler_params=pltpu.CompilerParams(
            needs_layout_passes=False, use_tc_tiling_on_sc=False))


def _tc_matmul(X, W):
    n, d_in = X.shape
    d_out = W.shape[0]

    def body(x_ref, w_ref, o_ref):
        o_ref[...] = lax.dot_general(
            x_ref[...], w_ref[...], (((1,), (1,)), ((), ())),
            preferred_element_type=jnp.float32)

    return pl.pallas_call(
        body,
        grid=(n // ROW_BLK,),
        in_specs=[
            pl.BlockSpec((ROW_BLK, d_in), lambda i: (i, 0)),
            pl.BlockSpec((d_out, d_in), lambda i: (0, 0)),
        ],
        out_specs=pl.BlockSpec((ROW_BLK, d_out), lambda i: (i, 0)),
        out_shape=jax.ShapeDtypeStruct((n, d_out), jnp.float32),
    )(X, W)


def _tc_mean_combine(esum_part, cnt_part):
    nseg = esum_part.shape[1]

    def body(e_ref, c_ref, o_ref):
        cnt = jnp.sum(c_ref[0] + c_ref[1], axis=-1) * (1.0 / LANES)
        e = e_ref[0] + e_ref[1]
        o_ref[...] = e / jnp.maximum(cnt, 1.0)[:, None]

    return pl.pallas_call(
        body,
        grid=(nseg // ROW_BLK,),
        in_specs=[
            pl.BlockSpec((NC, ROW_BLK, D), lambda i: (0, i, 0)),
            pl.BlockSpec((NC, ROW_BLK, LANES), lambda i: (0, i, 0)),
        ],
        out_specs=pl.BlockSpec((ROW_BLK, D), lambda i: (i, 0)),
        out_shape=jax.ShapeDtypeStruct((nseg, D), jnp.float32),
    )(esum_part, cnt_part)


def _tc_final(X2, xv_part, eps):
    n = X2.shape[0]

    def body(x2_ref, v_ref, eps_ref, o_ref):
        o_ref[...] = (1.0 + eps_ref[0]) * x2_ref[...] + v_ref[0] + v_ref[1]

    return pl.pallas_call(
        body,
        grid=(n // ROW_BLK,),
        in_specs=[
            pl.BlockSpec((ROW_BLK, D), lambda i: (i, 0)),
            pl.BlockSpec((NC, ROW_BLK, D), lambda i: (0, i, 0)),
            pl.BlockSpec(memory_space=pltpu.SMEM),
        ],
        out_specs=pl.BlockSpec((ROW_BLK, D), lambda i: (i, 0)),
        out_shape=jax.ShapeDtypeStruct((n, D), jnp.float32),
    )(X2, xv_part, eps)


def kernel(X, vertex, edges, W, eps):
    X2 = _tc_matmul(X, W)

    k1, k2 = 80, 80

    esum_part, cnt_part = _sc_gather_scatter_add(N_EDGES, True, k1)(
        X2, vertex.reshape(NW, PER_W // k1, k1),
        edges.reshape(NW, PER_W // k1, k1))
    Xe = _tc_mean_combine(esum_part, cnt_part)

    (xv_part,) = _sc_gather_scatter_add(N_NODES, False, k2, nbuf=3)(
        Xe, edges.reshape(NW, PER_W // k2, k2),
        vertex.reshape(NW, PER_W // k2, k2))

    Xout = _tc_final(X2, xv_part, eps)
    return (Xout, Xe)
